# baseline (speedup 1.0000x reference)
import functools

import jax
import jax.numpy as jnp
from jax import lax
from jax.experimental import pallas as pl
from jax.experimental.pallas import tpu as pltpu
import jax.experimental.pallas.tpu_sc as plsc

N = 10000
E = 320000
D = 128
NE = 4
NC = 2
NS = 16
K = 80
GA = 25
NGA = 5
G = 25
NG = 10
NPAD = 10240
ROWS_PER_TILE = NPAD // NS

_MESH = plsc.VectorSubcoreMesh(core_axis_name="c", subcore_axis_name="s")


def _burst3(table_hbm, src_v, dst_v, t0, rows, sems, ssems, acc):
    gds = [pltpu.async_copy(table_hbm.at[src_v.at[t0 + i]], rows[i], sems[i])
           for i in range(3)]
    sds = []
    for i in range(3):
        gds[i].wait()
        sds.append(pltpu.async_copy(rows[i], acc.at[dst_v.at[t0 + i]],
                                    ssems[i], add=True))
    for d in sds:
        d.wait()


def _group_rounds(table_hbm, src_v, dst_v, rows, sems, ssems, acc):
    def burst(q, carry):
        _burst3(table_hbm, src_v, dst_v, 3 * q, rows, sems, ssems, acc)
        return carry

    lax.fori_loop(0, GA // 3, burst, 0)
    pltpu.async_copy(table_hbm.at[src_v.at[GA - 1]], rows[0], sems[0]).wait()
    pltpu.sync_copy(rows[0], acc.at[dst_v.at[GA - 1]], add=True)


@functools.partial(
    pl.kernel,
    out_type=(jax.ShapeDtypeStruct((NC, NPAD, D), jnp.float32),
              jax.ShapeDtypeStruct((NC, NPAD, D), jnp.float32)),
    mesh=_MESH,
    scratch_types=[
        pltpu.VMEM((GA, K), jnp.int32),
        pltpu.VMEM((GA, K), jnp.int32),
        pltpu.VMEM((K, D), jnp.float32),
        pltpu.VMEM((K, D), jnp.float32),
        pltpu.VMEM((K, D), jnp.float32),
        pltpu.VMEM_SHARED((NPAD, D), jnp.float32),
        pltpu.SemaphoreType.DMA,
        pltpu.SemaphoreType.DMA,
        pltpu.SemaphoreType.DMA,
        pltpu.SemaphoreType.DMA,
        pltpu.SemaphoreType.DMA,
        pltpu.SemaphoreType.DMA,
    ],
)
def _sc_pass_a(x_hbm, srcb_hbm, dstb_hbm, ones_hbm, zeros_hbm,
               agg_hbm, deg_hbm,
               src_v, dst_v, rows0, rows1, rows2, acc,
               sg0, sg1, sg2, ss0, ss1, ss2):
    c = lax.axis_index("c")
    s = lax.axis_index("s")
    row0 = s * ROWS_PER_TILE
    w = s * NC + c
    pltpu.sync_copy(zeros_hbm.at[pl.ds(row0, ROWS_PER_TILE)],
                    acc.at[pl.ds(row0, ROWS_PER_TILE)])
    plsc.subcore_barrier()

    def group_x(g, carry):
        pltpu.sync_copy(srcb_hbm.at[w * NGA + g], src_v)
        pltpu.sync_copy(dstb_hbm.at[w * NGA + g], dst_v)
        _group_rounds(x_hbm, src_v, dst_v,
                      (rows0, rows1, rows2),
                      (sg0, sg1, sg2), (ss0, ss1, ss2), acc)
        return carry

    lax.fori_loop(0, NGA, group_x, 0)
    plsc.subcore_barrier()
    pltpu.sync_copy(acc.at[pl.ds(row0, ROWS_PER_TILE)],
                    agg_hbm.at[c, pl.ds(row0, ROWS_PER_TILE)])
    pltpu.sync_copy(zeros_hbm.at[pl.ds(row0, ROWS_PER_TILE)],
                    acc.at[pl.ds(row0, ROWS_PER_TILE)])
    pltpu.sync_copy(ones_hbm, rows0)
    plsc.subcore_barrier()

    def group_deg(g, carry):
        pltpu.sync_copy(dstb_hbm.at[w * NGA + g], dst_v)

        def pair_deg(p, carry2):
            d0 = pltpu.async_copy(rows0, acc.at[dst_v.at[2 * p]],
                                  sg0, add=True)
            d1 = pltpu.async_copy(rows0, acc.at[dst_v.at[2 * p + 1]],
                                  sg1, add=True)
            d0.wait()
            d1.wait()
            return carry2

        lax.fori_loop(0, GA // 2, pair_deg, 0)
        pltpu.async_copy(rows0, acc.at[dst_v.at[GA - 1]], sg0,
                         add=True).wait()
        return carry

    lax.fori_loop(0, NGA, group_deg, 0)
    plsc.subcore_barrier()
    pltpu.sync_copy(acc.at[pl.ds(row0, ROWS_PER_TILE)],
                    deg_hbm.at[c, pl.ds(row0, ROWS_PER_TILE)])


@functools.partial(
    pl.kernel,
    out_type=jax.ShapeDtypeStruct((NE * NPAD, D), jnp.float32),
    mesh=_MESH,
    scratch_types=[
        pltpu.VMEM((G, K), jnp.int32),
        pltpu.VMEM((G, K), jnp.int32),
        pltpu.VMEM((G, K), jnp.int32),
        pltpu.VMEM((G, K), jnp.int32),
        pltpu.VMEM((K, D), jnp.float32),
        pltpu.VMEM((K, D), jnp.float32),
        pltpu.VMEM((K, D), jnp.float32),
        pltpu.VMEM_SHARED((NPAD, D), jnp.float32),
        pltpu.SemaphoreType.DMA,
        pltpu.SemaphoreType.DMA,
        pltpu.SemaphoreType.DMA,
        pltpu.SemaphoreType.DMA,
        pltpu.SemaphoreType.DMA,
        pltpu.SemaphoreType.DMA,
        pltpu.SemaphoreType.DMA,
        pltpu.SemaphoreType.DMA,
    ],
)
def _sc_pass_b(h1_hbm, srcb4_hbm, dstb_hbm, zeros_hbm, out_hbm,
               src_a, dst_a, src_b, dst_b, rows0, rows1, rows2, acc,
               sg0, sg1, sg2, ss0, ss1, ss2, si0, si1):
    c = lax.axis_index("c")
    s = lax.axis_index("s")
    row0 = s * ROWS_PER_TILE
    rows = (rows0, rows1, rows2)
    gsems = (sg0, sg1, sg2)
    ssems = (ss0, ss1, ss2)
    for r in range(NE // NC):
        e = r * NC + c
        base = (e * NS + s) * NG
        dbase = s * NG
        pltpu.sync_copy(zeros_hbm.at[pl.ds(row0, ROWS_PER_TILE)],
                        acc.at[pl.ds(row0, ROWS_PER_TILE)])
        pltpu.sync_copy(srcb4_hbm.at[base], src_a)
        pltpu.sync_copy(dstb_hbm.at[dbase], dst_a)
        plsc.subcore_barrier()

        def qpair(q, carry):
            g = 2 * q
            pb0 = pltpu.async_copy(srcb4_hbm.at[base + g + 1], src_b, si0)
            pb1 = pltpu.async_copy(dstb_hbm.at[dbase + g + 1], dst_b, si1)
            _group_rounds(h1_hbm, src_a, dst_a, rows, gsems, ssems, acc)
            pb0.wait()
            pb1.wait()
            pa0 = pltpu.async_copy(srcb4_hbm.at[base + g + 2], src_a, si0)
            pa1 = pltpu.async_copy(dstb_hbm.at[dbase + g + 2], dst_a, si1)
            _group_rounds(h1_hbm, src_b, dst_b, rows, gsems, ssems, acc)
            pa0.wait()
            pa1.wait()
            return carry

        lax.fori_loop(0, NG // 2, qpair, 0)
        plsc.subcore_barrier()
        pltpu.sync_copy(acc.at[pl.ds(row0, ROWS_PER_TILE)],
                        out_hbm.at[pl.ds(e * NPAD + row0, ROWS_PER_TILE)])
        plsc.subcore_barrier()


BM = 2000
NB = N // BM


def _tc_layer0_mm(x_ref, aggp_ref, degp_ref, ws_ref, wn_ref, b_ref,
                  z_ref, mom_ref):
    agg = aggp_ref[0] + aggp_ref[1]
    deg = jnp.maximum(degp_ref[0, :, 0:1] + degp_ref[1, :, 0:1], 1.0)
    magg = agg / deg
    xb = x_ref[...]
    moms = []
    for e in range(NE):
        z = (jnp.dot(xb, ws_ref[e], preferred_element_type=jnp.float32)
             + jnp.dot(magg, wn_ref[e], preferred_element_type=jnp.float32)
             + b_ref[e][None, :])
        z_ref[e] = z
        moms.append(jnp.sum(z, axis=0, keepdims=True))
        moms.append(jnp.sum(z * z, axis=0, keepdims=True))
    mom_ref[0] = jnp.concatenate(moms[0::2] + moms[1::2], axis=0)


def _tc_layer1_mm(h1_ref, agg1_ref, degp_ref, ws_ref, wn_ref, b_ref,
                  z_ref, mom_ref):
    deg = jnp.maximum(degp_ref[0, :, 0:1] + degp_ref[1, :, 0:1], 1.0)
    moms = []
    for e in range(NE):
        magg = agg1_ref[e] / deg
        z = (jnp.dot(h1_ref[e], ws_ref[e], preferred_element_type=jnp.float32)
             + jnp.dot(magg, wn_ref[e], preferred_element_type=jnp.float32)
             + b_ref[e][None, :])
        z_ref[e] = z
        moms.append(jnp.sum(z, axis=0, keepdims=True))
        moms.append(jnp.sum(z * z, axis=0, keepdims=True))
    mom_ref[0] = jnp.concatenate(moms[0::2] + moms[1::2], axis=0)


def _tc_bn_relu(z_ref, mom_ref, gb_ref, out_ref):
    m = jnp.sum(mom_ref[...], axis=0)
    for e in range(NE):
        mu = m[e] / N
        var = m[NE + e] / N - mu * mu
        inv = gb_ref[e] * lax.rsqrt(var + 1e-5)
        h = inv[None, :] * (z_ref[e] - mu[None, :]) + gb_ref[NE + e][None, :]
        out_ref[e] = jnp.maximum(h, 0.0)


def _full(shape):
    return pl.BlockSpec(shape, lambda i: (0,) * len(shape))


def _rows3(lead):
    return pl.BlockSpec((lead, BM, D), lambda i: (0, i, 0))


def _layer0_mm(x, aggp, degp, ws, wn, bias):
    return pl.pallas_call(
        _tc_layer0_mm,
        grid=(NB,),
        in_specs=[
            pl.BlockSpec((BM, D), lambda i: (i, 0)),
            pl.BlockSpec((NC, BM, D), lambda i: (0, i, 0)),
            pl.BlockSpec((NC, BM, D), lambda i: (0, i, 0)),
            _full((NE, D, D)),
            _full((NE, D, D)),
            _full((8, D)),
        ],
        out_specs=[_rows3(NE), pl.BlockSpec((1, 8, D), lambda i: (i, 0, 0))],
        out_shape=[
            jax.ShapeDtypeStruct((NE, N, D), jnp.float32),
            jax.ShapeDtypeStruct((NB, 8, D), jnp.float32),
        ],
    )(x, aggp, degp, ws, wn, bias)


def _layer1_mm(h1, agg1, degp, ws, wn, bias):
    return pl.pallas_call(
        _tc_layer1_mm,
        grid=(NB,),
        in_specs=[
            _rows3(NE),
            _rows3(NE),
            pl.BlockSpec((NC, BM, D), lambda i: (0, i, 0)),
            _full((NE, D, D)),
            _full((NE, D, D)),
            _full((8, D)),
        ],
        out_specs=[_rows3(NE), pl.BlockSpec((1, 8, D), lambda i: (i, 0, 0))],
        out_shape=[
            jax.ShapeDtypeStruct((NE, N, D), jnp.float32),
            jax.ShapeDtypeStruct((NB, 8, D), jnp.float32),
        ],
    )(h1, agg1, degp, ws, wn, bias)


def _bn_relu(z, mom, gb):
    return pl.pallas_call(
        _tc_bn_relu,
        grid=(NB,),
        in_specs=[_rows3(NE), _full((NB, 8, D)), _full((8, D))],
        out_specs=_rows3(NE),
        out_shape=jax.ShapeDtypeStruct((NE, N, D), jnp.float32),
    )(z, mom, gb)


def kernel(x, edge_index, Ws, Wn, b, gamma, beta):
    src = edge_index[0].astype(jnp.int32)
    dst = edge_index[1].astype(jnp.int32)
    srcb_a = src.reshape(NC * NS * NGA, GA, K)
    dstb_a = dst.reshape(NC * NS * NGA, GA, K)
    offs = (jnp.arange(NE, dtype=jnp.int32) * N)[:, None]
    garbage = jnp.zeros((1, G, K), jnp.int32)
    srcb4 = jnp.concatenate(
        [(src[None, :] + offs).reshape(NE * NS * NG, G, K), garbage], axis=0)
    dstb = jnp.concatenate([dst.reshape(NS * NG, G, K), garbage], axis=0)

    ones_k = jnp.ones((K, D), jnp.float32)
    zeros_d = jnp.zeros((NPAD, D), jnp.float32)

    pad4 = jnp.zeros((NE, D), jnp.float32)
    bias0 = jnp.concatenate([b[:, 0], pad4], axis=0)
    bias1 = jnp.concatenate([b[:, 1], pad4], axis=0)
    gb0 = jnp.concatenate([gamma[:, 0], beta[:, 0]], axis=0)
    gb1 = jnp.concatenate([gamma[:, 1], beta[:, 1]], axis=0)

    aggp, degp = _sc_pass_a(x, srcb_a, dstb_a, ones_k, zeros_d)
    z0, mom0 = _layer0_mm(x, aggp, degp, Ws[:, 0], Wn[:, 0], bias0)
    h1 = _bn_relu(z0, mom0, gb0)

    agg1 = _sc_pass_b(h1.reshape(NE * N, D), srcb4, dstb, zeros_d)
    z1, mom1 = _layer1_mm(h1, agg1.reshape(NE, NPAD, D), degp,
                          Ws[:, 1], Wn[:, 1], bias1)
    h2 = _bn_relu(z1, mom1, gb1)

    return jnp.transpose(h2, (1, 2, 0))


# speedup vs baseline: 1.0083x; 1.0083x over previous
import functools

import jax
import jax.numpy as jnp
from jax import lax
from jax.experimental import pallas as pl
from jax.experimental.pallas import tpu as pltpu
import jax.experimental.pallas.tpu_sc as plsc

N = 10000
E = 320000
D = 128
NE = 4
NC = 2
NS = 16
K = 80
GA = 25
NGA = 5
G = 25
NG = 10
NPAD = 10240
ROWS_PER_TILE = NPAD // NS

_MESH = plsc.VectorSubcoreMesh(core_axis_name="c", subcore_axis_name="s")


def _burst4(table_hbm, src_v, dst_v, t0, rows, sems, ssems, acc):
    gds = [pltpu.async_copy(table_hbm.at[src_v.at[t0 + i]], rows[i], sems[i])
           for i in range(4)]
    sds = []
    for i in range(4):
        gds[i].wait()
        sds.append(pltpu.async_copy(rows[i], acc.at[dst_v.at[t0 + i]],
                                    ssems[i], add=True))
    for d in sds:
        d.wait()


def _group_rounds(table_hbm, src_v, dst_v, rows, sems, ssems, acc):
    def burst(q, carry):
        _burst4(table_hbm, src_v, dst_v, 4 * q, rows, sems, ssems, acc)
        return carry

    lax.fori_loop(0, GA // 4, burst, 0)
    pltpu.async_copy(table_hbm.at[src_v.at[GA - 1]], rows[0], sems[0]).wait()
    pltpu.sync_copy(rows[0], acc.at[dst_v.at[GA - 1]], add=True)


@functools.partial(
    pl.kernel,
    out_type=(jax.ShapeDtypeStruct((NC, NPAD, D), jnp.float32),
              jax.ShapeDtypeStruct((NC, NPAD, D), jnp.float32)),
    mesh=_MESH,
    scratch_types=[
        pltpu.VMEM((GA, K), jnp.int32),
        pltpu.VMEM((GA, K), jnp.int32),
        pltpu.VMEM((K, D), jnp.float32),
        pltpu.VMEM((K, D), jnp.float32),
        pltpu.VMEM((K, D), jnp.float32),
        pltpu.VMEM((K, D), jnp.float32),
        pltpu.VMEM_SHARED((NPAD, D), jnp.float32),
        pltpu.SemaphoreType.DMA,
        pltpu.SemaphoreType.DMA,
        pltpu.SemaphoreType.DMA,
        pltpu.SemaphoreType.DMA,
        pltpu.SemaphoreType.DMA,
        pltpu.SemaphoreType.DMA,
        pltpu.SemaphoreType.DMA,
        pltpu.SemaphoreType.DMA,
    ],
)
def _sc_pass_a(x_hbm, srcb_hbm, dstb_hbm, ones_hbm, zeros_hbm,
               agg_hbm, deg_hbm,
               src_v, dst_v, rows0, rows1, rows2, rows3, acc,
               sg0, sg1, sg2, sg3, ss0, ss1, ss2, ss3):
    c = lax.axis_index("c")
    s = lax.axis_index("s")
    row0 = s * ROWS_PER_TILE
    w = s * NC + c
    pltpu.sync_copy(zeros_hbm.at[pl.ds(row0, ROWS_PER_TILE)],
                    acc.at[pl.ds(row0, ROWS_PER_TILE)])
    plsc.subcore_barrier()

    def group_x(g, carry):
        pltpu.sync_copy(srcb_hbm.at[w * NGA + g], src_v)
        pltpu.sync_copy(dstb_hbm.at[w * NGA + g], dst_v)
        _group_rounds(x_hbm, src_v, dst_v,
                      (rows0, rows1, rows2, rows3),
                      (sg0, sg1, sg2, sg3), (ss0, ss1, ss2, ss3), acc)
        return carry

    lax.fori_loop(0, NGA, group_x, 0)
    plsc.subcore_barrier()
    pltpu.sync_copy(acc.at[pl.ds(row0, ROWS_PER_TILE)],
                    agg_hbm.at[c, pl.ds(row0, ROWS_PER_TILE)])
    pltpu.sync_copy(zeros_hbm.at[pl.ds(row0, ROWS_PER_TILE)],
                    acc.at[pl.ds(row0, ROWS_PER_TILE)])
    pltpu.sync_copy(ones_hbm, rows0)
    plsc.subcore_barrier()

    def group_deg(g, carry):
        pltpu.sync_copy(dstb_hbm.at[w * NGA + g], dst_v)

        def pair_deg(p, carry2):
            d0 = pltpu.async_copy(rows0, acc.at[dst_v.at[2 * p]],
                                  sg0, add=True)
            d1 = pltpu.async_copy(rows0, acc.at[dst_v.at[2 * p + 1]],
                                  sg1, add=True)
            d0.wait()
            d1.wait()
            return carry2

        lax.fori_loop(0, GA // 2, pair_deg, 0)
        pltpu.async_copy(rows0, acc.at[dst_v.at[GA - 1]], sg0,
                         add=True).wait()
        return carry

    lax.fori_loop(0, NGA, group_deg, 0)
    plsc.subcore_barrier()
    pltpu.sync_copy(acc.at[pl.ds(row0, ROWS_PER_TILE)],
                    deg_hbm.at[c, pl.ds(row0, ROWS_PER_TILE)])


@functools.partial(
    pl.kernel,
    out_type=jax.ShapeDtypeStruct((NE * NPAD, D), jnp.float32),
    mesh=_MESH,
    scratch_types=[
        pltpu.VMEM((G, K), jnp.int32),
        pltpu.VMEM((G, K), jnp.int32),
        pltpu.VMEM((K, D), jnp.float32),
        pltpu.VMEM((K, D), jnp.float32),
        pltpu.VMEM((K, D), jnp.float32),
        pltpu.VMEM((K, D), jnp.float32),
        pltpu.VMEM_SHARED((NPAD, D), jnp.float32),
        pltpu.SemaphoreType.DMA,
        pltpu.SemaphoreType.DMA,
        pltpu.SemaphoreType.DMA,
        pltpu.SemaphoreType.DMA,
        pltpu.SemaphoreType.DMA,
        pltpu.SemaphoreType.DMA,
        pltpu.SemaphoreType.DMA,
        pltpu.SemaphoreType.DMA,
    ],
)
def _sc_pass_b(h1_hbm, srcb4_hbm, dstb_hbm, zeros_hbm, out_hbm,
               src_v, dst_v, rows0, rows1, rows2, rows3, acc,
               sg0, sg1, sg2, sg3, ss0, ss1, ss2, ss3):
    c = lax.axis_index("c")
    s = lax.axis_index("s")
    row0 = s * ROWS_PER_TILE
    for r in range(NE // NC):
        e = r * NC + c
        pltpu.sync_copy(zeros_hbm.at[pl.ds(row0, ROWS_PER_TILE)],
                        acc.at[pl.ds(row0, ROWS_PER_TILE)])
        plsc.subcore_barrier()

        def group(g, carry):
            pltpu.sync_copy(srcb4_hbm.at[(e * NS + s) * NG + g], src_v)
            pltpu.sync_copy(dstb_hbm.at[s * NG + g], dst_v)
            _group_rounds(h1_hbm, src_v, dst_v,
                          (rows0, rows1, rows2, rows3),
                          (sg0, sg1, sg2, sg3), (ss0, ss1, ss2, ss3), acc)
            return carry

        lax.fori_loop(0, NG, group, 0)
        plsc.subcore_barrier()
        pltpu.sync_copy(acc.at[pl.ds(row0, ROWS_PER_TILE)],
                        out_hbm.at[pl.ds(e * NPAD + row0, ROWS_PER_TILE)])
        plsc.subcore_barrier()


BM = 2000
NB = N // BM


def _tc_layer0_mm(x_ref, aggp_ref, degp_ref, ws_ref, wn_ref, b_ref,
                  z_ref, mom_ref):
    agg = aggp_ref[0] + aggp_ref[1]
    deg = jnp.maximum(degp_ref[0, :, 0:1] + degp_ref[1, :, 0:1], 1.0)
    magg = agg / deg
    xb = x_ref[...]
    moms = []
    for e in range(NE):
        z = (jnp.dot(xb, ws_ref[e], preferred_element_type=jnp.float32)
             + jnp.dot(magg, wn_ref[e], preferred_element_type=jnp.float32)
             + b_ref[e][None, :])
        z_ref[e] = z
        moms.append(jnp.sum(z, axis=0, keepdims=True))
        moms.append(jnp.sum(z * z, axis=0, keepdims=True))
    mom_ref[0] = jnp.concatenate(moms[0::2] + moms[1::2], axis=0)


def _tc_layer1_mm(h1_ref, agg1_ref, degp_ref, ws_ref, wn_ref, b_ref,
                  z_ref, mom_ref):
    deg = jnp.maximum(degp_ref[0, :, 0:1] + degp_ref[1, :, 0:1], 1.0)
    moms = []
    for e in range(NE):
        magg = agg1_ref[e] / deg
        z = (jnp.dot(h1_ref[e], ws_ref[e], preferred_element_type=jnp.float32)
             + jnp.dot(magg, wn_ref[e], preferred_element_type=jnp.float32)
             + b_ref[e][None, :])
        z_ref[e] = z
        moms.append(jnp.sum(z, axis=0, keepdims=True))
        moms.append(jnp.sum(z * z, axis=0, keepdims=True))
    mom_ref[0] = jnp.concatenate(moms[0::2] + moms[1::2], axis=0)


def _tc_bn_relu(z_ref, mom_ref, gb_ref, out_ref):
    m = jnp.sum(mom_ref[...], axis=0)
    for e in range(NE):
        mu = m[e] / N
        var = m[NE + e] / N - mu * mu
        inv = gb_ref[e] * lax.rsqrt(var + 1e-5)
        h = inv[None, :] * (z_ref[e] - mu[None, :]) + gb_ref[NE + e][None, :]
        out_ref[e] = jnp.maximum(h, 0.0)


def _full(shape):
    return pl.BlockSpec(shape, lambda i: (0,) * len(shape))


def _rows3(lead):
    return pl.BlockSpec((lead, BM, D), lambda i: (0, i, 0))


def _layer0_mm(x, aggp, degp, ws, wn, bias):
    return pl.pallas_call(
        _tc_layer0_mm,
        grid=(NB,),
        in_specs=[
            pl.BlockSpec((BM, D), lambda i: (i, 0)),
            pl.BlockSpec((NC, BM, D), lambda i: (0, i, 0)),
            pl.BlockSpec((NC, BM, D), lambda i: (0, i, 0)),
            _full((NE, D, D)),
            _full((NE, D, D)),
            _full((8, D)),
        ],
        out_specs=[_rows3(NE), pl.BlockSpec((1, 8, D), lambda i: (i, 0, 0))],
        out_shape=[
            jax.ShapeDtypeStruct((NE, N, D), jnp.float32),
            jax.ShapeDtypeStruct((NB, 8, D), jnp.float32),
        ],
    )(x, aggp, degp, ws, wn, bias)


def _layer1_mm(h1, agg1, degp, ws, wn, bias):
    return pl.pallas_call(
        _tc_layer1_mm,
        grid=(NB,),
        in_specs=[
            _rows3(NE),
            _rows3(NE),
            pl.BlockSpec((NC, BM, D), lambda i: (0, i, 0)),
            _full((NE, D, D)),
            _full((NE, D, D)),
            _full((8, D)),
        ],
        out_specs=[_rows3(NE), pl.BlockSpec((1, 8, D), lambda i: (i, 0, 0))],
        out_shape=[
            jax.ShapeDtypeStruct((NE, N, D), jnp.float32),
            jax.ShapeDtypeStruct((NB, 8, D), jnp.float32),
        ],
    )(h1, agg1, degp, ws, wn, bias)


def _bn_relu(z, mom, gb):
    return pl.pallas_call(
        _tc_bn_relu,
        grid=(NB,),
        in_specs=[_rows3(NE), _full((NB, 8, D)), _full((8, D))],
        out_specs=_rows3(NE),
        out_shape=jax.ShapeDtypeStruct((NE, N, D), jnp.float32),
    )(z, mom, gb)


def kernel(x, edge_index, Ws, Wn, b, gamma, beta):
    src = edge_index[0].astype(jnp.int32)
    dst = edge_index[1].astype(jnp.int32)
    srcb_a = src.reshape(NC * NS * NGA, GA, K)
    dstb_a = dst.reshape(NC * NS * NGA, GA, K)
    offs = (jnp.arange(NE, dtype=jnp.int32) * N)[:, None]
    srcb4 = (src[None, :] + offs).reshape(NE * NS * NG, G, K)
    dstb = dst.reshape(NS * NG, G, K)

    ones_k = jnp.ones((K, D), jnp.float32)
    zeros_d = jnp.zeros((NPAD, D), jnp.float32)

    pad4 = jnp.zeros((NE, D), jnp.float32)
    bias0 = jnp.concatenate([b[:, 0], pad4], axis=0)
    bias1 = jnp.concatenate([b[:, 1], pad4], axis=0)
    gb0 = jnp.concatenate([gamma[:, 0], beta[:, 0]], axis=0)
    gb1 = jnp.concatenate([gamma[:, 1], beta[:, 1]], axis=0)

    aggp, degp = _sc_pass_a(x, srcb_a, dstb_a, ones_k, zeros_d)
    z0, mom0 = _layer0_mm(x, aggp, degp, Ws[:, 0], Wn[:, 0], bias0)
    h1 = _bn_relu(z0, mom0, gb0)

    agg1 = _sc_pass_b(h1.reshape(NE * N, D), srcb4, dstb, zeros_d)
    z1, mom1 = _layer1_mm(h1, agg1.reshape(NE, NPAD, D), degp,
                          Ws[:, 1], Wn[:, 1], bias1)
    h2 = _bn_relu(z1, mom1, gb1)

    return jnp.transpose(h2, (1, 2, 0))
